# Initial kernel scaffold; baseline (speedup 1.0000x reference)
import jax
import jax.numpy as jnp
from jax.experimental import pallas as pl


def kernel(x, edge_index, batch, W1, b1, W2, b2, W3, b3, gamma, beta):
    raise NotImplementedError("write your pallas kernel here")


# speedup vs baseline: 10.5068x; 10.5068x over previous
import functools

import jax
import jax.numpy as jnp
from jax import lax
from jax.experimental import pallas as pl
from jax.experimental.pallas import tpu as pltpu
from jax.experimental.pallas import tpu_sc as plsc

N_NODES = 10000
N_EDGES = 320000
D = 128
N_GRAPHS = 256

NC = 2
NS = 16
NW = NC * NS
CHUNK = 125
NCHUNK = N_EDGES // (NW * CHUNK)
BLK = 16
NBLK = NCHUNK // BLK
N_PAD = 10240
ROWS_PER_TILE = N_PAD // NS
ZCH = 32


def _sc_agg_kernel(x_hbm, src_hbm, dst_hbm, out_hbm,
                   sblk, dblk, r0, r1, zbuf, agg_sh,
                   sem_g0, sem_g1, sem_s, sem_d):
    c = lax.axis_index("c")
    s = lax.axis_index("s")
    row0 = s * ROWS_PER_TILE

    zeros16 = jnp.zeros((16,), jnp.float32)

    def _zero_body(t, _):
        zbuf[t // 8, pl.ds((t % 8) * 16, 16)] = zeros16
        return 0

    lax.fori_loop(0, ZCH * 8, _zero_body, 0)
    for m in range(ROWS_PER_TILE // ZCH):
        pltpu.sync_copy(zbuf, agg_sh.at[pl.ds(row0 + m * ZCH, ZCH)])
    plsc.subcore_barrier()

    def _blk_body(blk, _):
        pltpu.sync_copy(src_hbm.at[c, s, pl.ds(blk * BLK, BLK)], sblk)
        pltpu.sync_copy(dst_hbm.at[c, s, pl.ds(blk * BLK, BLK)], dblk)
        pltpu.async_copy(x_hbm.at[sblk.at[0]], r0, sem_g0)
        pltpu.async_copy(x_hbm.at[sblk.at[1]], r1, sem_g1)

        def _pair(i, _):
            j0 = 2 * i
            pltpu.make_async_copy(x_hbm.at[sblk.at[j0]], r0, sem_g0).wait()
            pltpu.async_copy(r0, agg_sh.at[dblk.at[j0]], sem_s, add=True).wait()

            @pl.when(j0 + 2 < BLK)
            def _():
                pltpu.async_copy(x_hbm.at[sblk.at[j0 + 2]], r0, sem_g0)

            pltpu.make_async_copy(x_hbm.at[sblk.at[j0 + 1]], r1, sem_g1).wait()
            pltpu.async_copy(r1, agg_sh.at[dblk.at[j0 + 1]], sem_s, add=True).wait()

            @pl.when(j0 + 3 < BLK)
            def _():
                pltpu.async_copy(x_hbm.at[sblk.at[j0 + 3]], r1, sem_g1)

            return 0

        lax.fori_loop(0, BLK // 2, _pair, 0)
        return 0

    lax.fori_loop(0, NBLK, _blk_body, 0)
    plsc.subcore_barrier()

    pltpu.async_copy(
        agg_sh.at[pl.ds(row0, ROWS_PER_TILE)],
        out_hbm.at[c, pl.ds(row0, ROWS_PER_TILE)],
        sem_d,
    ).wait()


@jax.jit
def _sc_aggregate(x, src, dst):
    run = pl.kernel(
        _sc_agg_kernel,
        mesh=plsc.VectorSubcoreMesh(core_axis_name="c", subcore_axis_name="s"),
        out_type=jax.ShapeDtypeStruct((NC, N_PAD, D), jnp.float32),
        scratch_types=[
            pltpu.VMEM((BLK, CHUNK), jnp.int32),
            pltpu.VMEM((BLK, CHUNK), jnp.int32),
            pltpu.VMEM((CHUNK, D), jnp.float32),
            pltpu.VMEM((CHUNK, D), jnp.float32),
            pltpu.VMEM((ZCH, D), jnp.float32),
            pltpu.VMEM_SHARED((N_PAD, D), jnp.float32),
            pltpu.SemaphoreType.DMA,
            pltpu.SemaphoreType.DMA,
            pltpu.SemaphoreType.DMA,
            pltpu.SemaphoreType.DMA,
        ],
    )
    return run(x, src, dst)


def _tc_body(x_ref, agg_ref, batch_ref, w1_ref, b1_ref, w2_ref, b2_ref,
             w3_ref, b3_ref, gamma_ref, beta_ref, out_ref):
    h = x_ref[...] + agg_ref[0] + agg_ref[1]
    h = jnp.maximum(jnp.dot(h, w1_ref[...],
                            preferred_element_type=jnp.float32) + b1_ref[...], 0.0)
    h = jnp.maximum(jnp.dot(h, w2_ref[...],
                            preferred_element_type=jnp.float32) + b2_ref[...], 0.0)
    h = jnp.dot(h, w3_ref[...], preferred_element_type=jnp.float32) + b3_ref[...]
    h = jnp.maximum(h, 0.0)
    mean = jnp.mean(h, axis=0, keepdims=True)
    var = jnp.mean((h - mean) ** 2, axis=0, keepdims=True)
    h = (h - mean) / jnp.sqrt(var + 1e-5) * gamma_ref[...] + beta_ref[...]
    onehot = (batch_ref[...] ==
              lax.broadcasted_iota(jnp.int32, (N_NODES, N_GRAPHS), 1)
              ).astype(jnp.float32)
    out_ref[...] = lax.dot_general(
        onehot, h, (((0,), (0,)), ((), ())),
        preferred_element_type=jnp.float32)


@jax.jit
def _tc_tail(x, agg, batch2d, W1, b1, W2, b2, W3, b3, gamma, beta):
    return pl.pallas_call(
        _tc_body,
        out_shape=jax.ShapeDtypeStruct((N_GRAPHS, D), jnp.float32),
    )(x, agg, batch2d, W1, b1, W2, b2, W3, b3, gamma, beta)


def kernel(x, edge_index, batch, W1, b1, W2, b2, W3, b3, gamma, beta):
    src = edge_index[0].astype(jnp.int32).reshape(NC, NS, NCHUNK, CHUNK)
    dst = edge_index[1].astype(jnp.int32).reshape(NC, NS, NCHUNK, CHUNK)
    agg = _sc_aggregate(x, src, dst)[:, :N_NODES]
    batch2d = batch.astype(jnp.int32).reshape(N_NODES, 1)
    return _tc_tail(x, agg, batch2d,
                    W1, b1.reshape(1, D), W2, b2.reshape(1, D),
                    W3, b3.reshape(1, D), gamma.reshape(1, D),
                    beta.reshape(1, D))


# speedup vs baseline: 10.9561x; 1.0428x over previous
import functools

import jax
import jax.numpy as jnp
from jax import lax
from jax.experimental import pallas as pl
from jax.experimental.pallas import tpu as pltpu
from jax.experimental.pallas import tpu_sc as plsc

N_NODES = 10000
N_EDGES = 320000
D = 128
N_GRAPHS = 256

NC = 2
NS = 16
NW = NC * NS
CHUNK = 125
NCHUNK = N_EDGES // (NW * CHUNK)
BLK = 16
NBLK = NCHUNK // BLK
N_PAD = 10240
ROWS_PER_TILE = N_PAD // NS
ZCH = 32


def _sc_agg_kernel(x_hbm, src_hbm, dst_hbm, out_hbm,
                   sblk, dblk, r0, r1, zbuf, agg_sh,
                   sem_g0, sem_g1, sem_s, sem_d):
    c = lax.axis_index("c")
    s = lax.axis_index("s")
    row0 = s * ROWS_PER_TILE

    zeros16 = jnp.zeros((16,), jnp.float32)

    def _zero_body(t, _):
        zbuf[t // 8, pl.ds((t % 8) * 16, 16)] = zeros16
        return 0

    lax.fori_loop(0, ZCH * 8, _zero_body, 0)
    for m in range(ROWS_PER_TILE // ZCH):
        pltpu.sync_copy(zbuf, agg_sh.at[pl.ds(row0 + m * ZCH, ZCH)])
    plsc.subcore_barrier()

    def _blk_body(blk, _):
        pltpu.sync_copy(src_hbm.at[c, s, pl.ds(blk * BLK, BLK)], sblk)
        pltpu.sync_copy(dst_hbm.at[c, s, pl.ds(blk * BLK, BLK)], dblk)
        pltpu.async_copy(x_hbm.at[sblk.at[0]], r0, sem_g0)
        pltpu.async_copy(x_hbm.at[sblk.at[1]], r1, sem_g1)

        def _pair(i, _):
            j0 = 2 * i
            pltpu.make_async_copy(x_hbm.at[sblk.at[j0]], r0, sem_g0).wait()
            pltpu.async_copy(r0, agg_sh.at[dblk.at[j0]], sem_s, add=True).wait()

            @pl.when(j0 + 2 < BLK)
            def _():
                pltpu.async_copy(x_hbm.at[sblk.at[j0 + 2]], r0, sem_g0)

            pltpu.make_async_copy(x_hbm.at[sblk.at[j0 + 1]], r1, sem_g1).wait()
            pltpu.async_copy(r1, agg_sh.at[dblk.at[j0 + 1]], sem_s, add=True).wait()

            @pl.when(j0 + 3 < BLK)
            def _():
                pltpu.async_copy(x_hbm.at[sblk.at[j0 + 3]], r1, sem_g1)

            return 0

        lax.fori_loop(0, BLK // 2, _pair, 0)
        return 0

    lax.fori_loop(0, NBLK, _blk_body, 0)
    plsc.subcore_barrier()

    pltpu.async_copy(
        agg_sh.at[pl.ds(row0, ROWS_PER_TILE)],
        out_hbm.at[c, pl.ds(row0, ROWS_PER_TILE)],
        sem_d,
    ).wait()


@jax.jit
def _sc_aggregate(x, src, dst):
    run = pl.kernel(
        _sc_agg_kernel,
        mesh=plsc.VectorSubcoreMesh(core_axis_name="c", subcore_axis_name="s"),
        out_type=jax.ShapeDtypeStruct((NC, N_PAD, D), jnp.float32),
        scratch_types=[
            pltpu.VMEM((BLK, CHUNK), jnp.int32),
            pltpu.VMEM((BLK, CHUNK), jnp.int32),
            pltpu.VMEM((CHUNK, D), jnp.float32),
            pltpu.VMEM((CHUNK, D), jnp.float32),
            pltpu.VMEM((ZCH, D), jnp.float32),
            pltpu.VMEM_SHARED((N_PAD, D), jnp.float32),
            pltpu.SemaphoreType.DMA,
            pltpu.SemaphoreType.DMA,
            pltpu.SemaphoreType.DMA,
            pltpu.SemaphoreType.DMA,
        ],
    )
    return run(x, src, dst)


def _tc_body(x_ref, agg_ref, batch_ref, w1_ref, b1_ref, w2_ref, b2_ref,
             w3_ref, b3_ref, gamma_ref, beta_ref, out_ref):
    h = x_ref[...] + agg_ref[0] + agg_ref[1]
    h = jnp.maximum(jnp.dot(h, w1_ref[...],
                            preferred_element_type=jnp.float32) + b1_ref[...], 0.0)
    h = jnp.maximum(jnp.dot(h, w2_ref[...],
                            preferred_element_type=jnp.float32) + b2_ref[...], 0.0)
    h = jnp.dot(h, w3_ref[...], preferred_element_type=jnp.float32) + b3_ref[...]
    h = jnp.maximum(h, 0.0)
    mean = jnp.mean(h, axis=0, keepdims=True)
    var = jnp.mean((h - mean) ** 2, axis=0, keepdims=True)
    h = (h - mean) / jnp.sqrt(var + 1e-5) * gamma_ref[...] + beta_ref[...]
    onehot = (batch_ref[...] ==
              lax.broadcasted_iota(jnp.int32, (N_NODES, N_GRAPHS), 1)
              ).astype(jnp.float32)
    out_ref[...] = lax.dot_general(
        onehot, h, (((0,), (0,)), ((), ())),
        preferred_element_type=jnp.float32)


def _tc_tail(x, agg, batch2d, W1, b1, W2, b2, W3, b3, gamma, beta):
    full2 = pl.BlockSpec((N_NODES, D), lambda i: (0, 0))
    wspec = pl.BlockSpec((D, D), lambda i: (0, 0))
    bspec = pl.BlockSpec((1, D), lambda i: (0, 0))
    return pl.pallas_call(
        _tc_body,
        out_shape=jax.ShapeDtypeStruct((N_GRAPHS, D), jnp.float32),
        grid=(1,),
        in_specs=[
            full2,
            pl.BlockSpec((NC, N_NODES, D), lambda i: (0, 0, 0)),
            pl.BlockSpec((N_NODES, 1), lambda i: (0, 0)),
            wspec, bspec, wspec, bspec, wspec, bspec, bspec, bspec,
        ],
        out_specs=pl.BlockSpec((N_GRAPHS, D), lambda i: (0, 0)),
    )(x, agg, batch2d, W1, b1, W2, b2, W3, b3, gamma, beta)


@jax.jit
def kernel(x, edge_index, batch, W1, b1, W2, b2, W3, b3, gamma, beta):
    src = edge_index[0].astype(jnp.int32).reshape(NC, NS, NCHUNK, CHUNK)
    dst = edge_index[1].astype(jnp.int32).reshape(NC, NS, NCHUNK, CHUNK)
    agg = _sc_aggregate(x, src, dst)
    batch2d = batch.astype(jnp.int32).reshape(N_NODES, 1)
    return _tc_tail(x, agg, batch2d,
                    W1, b1.reshape(1, D), W2, b2.reshape(1, D),
                    W3, b3.reshape(1, D), gamma.reshape(1, D),
                    beta.reshape(1, D))
